# baseline (speedup 1.0000x reference)
import functools

import jax
import jax.numpy as jnp
from jax import lax
from jax.experimental import pallas as pl
from jax.experimental.pallas import tpu as pltpu
from jax.experimental.pallas import tpu_sc as plsc

POOL_SIZE = 1000000
LATENT_DIM = 64
BATCH = 16384

_info = plsc.get_sparse_core_info()
_NC, _NS, _L = _info.num_cores, _info.num_subcores, _info.num_lanes
_NW = _NC * _NS
_BPW = BATCH // _NW
_G = _L
_NG = _BPW // _G

_mesh = plsc.VectorSubcoreMesh(core_axis_name="c", subcore_axis_name="s")


@functools.partial(
    pl.kernel,
    mesh=_mesh,
    out_type=jax.ShapeDtypeStruct((BATCH, LATENT_DIM), jnp.float32),
    scratch_types=[
        pltpu.VMEM((_BPW,), jnp.int32),
        pltpu.VMEM((_G, 8, LATENT_DIM), jnp.float32),
        pltpu.VMEM((_G, 8, LATENT_DIM), jnp.float32),
        pltpu.VMEM((_G, LATENT_DIM), jnp.float32),
        pltpu.VMEM((_G, LATENT_DIM), jnp.float32),
        pltpu.SemaphoreType.DMA,
        pltpu.SemaphoreType.DMA,
        pltpu.SemaphoreType.DMA,
        pltpu.SemaphoreType.DMA,
    ],
    compiler_params=pltpu.CompilerParams(needs_layout_passes=False),
)
def _gather_sc(table_hbm, idx_hbm, out_hbm, idx_v, tiles0_v, tiles1_v,
               sel0_v, sel1_v, gsem0, gsem1, wsem0, wsem1):
    wid = lax.axis_index("s") * _NC + lax.axis_index("c")
    base = wid * _BPW
    pltpu.sync_copy(idx_hbm.at[pl.ds(base, _BPW)], idx_v)

    lane = lax.iota(jnp.int32, _L)
    tile_rings = (tiles0_v, tiles1_v)
    sel_rings = (sel0_v, sel1_v)
    gsems = (gsem0, gsem1)
    wsems = (wsem0, wsem1)

    def fire(g, ring):
        idx16 = idx_v[pl.ds(g * _G, _G)]
        for s in range(_G):
            i = jnp.sum(jnp.where(lane == s, idx16, 0))
            pltpu.async_copy(table_hbm.at[pl.ds((i >> 3) * 8, 8)],
                             tile_rings[ring].at[s], gsems[ring])

    def process(g, h, ring):
        for s in range(_G):
            pltpu.make_async_copy(table_hbm.at[pl.ds(0, 8)],
                                  tile_rings[ring].at[s], gsems[ring]).wait()
        @pl.when(h >= 1)
        def _():
            pltpu.make_async_copy(sel_rings[ring], out_hbm.at[pl.ds(0, _G)],
                                  wsems[ring]).wait()
        sub = idx_v[pl.ds(g * _G, _G)] & 7

        def col_body(c, carry):
            cvec = jnp.full((_L,), c, jnp.int32)
            vals = plsc.load_gather(tile_rings[ring], [lane, sub, cvec])
            plsc.store_scatter(sel_rings[ring], [lane, cvec], vals)
            return carry

        lax.fori_loop(0, LATENT_DIM, col_body, 0, unroll=8)
        pltpu.async_copy(sel_rings[ring], out_hbm.at[pl.ds(base + g * _G, _G)],
                         wsems[ring])

    fire(0, 0)

    def body2(h, carry):
        g0 = 2 * h
        fire(g0 + 1, 1)
        process(g0, h, 0)

        @pl.when(h < _NG // 2 - 1)
        def _():
            fire(g0 + 2, 0)

        process(g0 + 1, h, 1)
        return carry

    lax.fori_loop(0, _NG // 2, body2, 0)

    for ring in range(2):
        pltpu.make_async_copy(sel_rings[ring], out_hbm.at[pl.ds(0, _G)],
                              wsems[ring]).wait()


def kernel(indices, latents):
    return _gather_sc(latents, indices.astype(jnp.int32))


# speedup vs baseline: 1.2440x; 1.2440x over previous
import functools

import jax
import jax.numpy as jnp
from jax import lax
from jax.experimental import pallas as pl
from jax.experimental.pallas import tpu as pltpu
from jax.experimental.pallas import tpu_sc as plsc

POOL_SIZE = 1000000
LATENT_DIM = 64
BATCH = 16384

_info = plsc.get_sparse_core_info()
_NC, _NS, _L = _info.num_cores, _info.num_subcores, _info.num_lanes
_NW = _NC * _NS
_SB = 128
_NSB = POOL_SIZE // _SB
_TAIL = POOL_SIZE - _NSB * _SB
_FULL_K = _NSB // _NW
_RING = 4

_mesh = plsc.VectorSubcoreMesh(core_axis_name="c", subcore_axis_name="s")


@functools.partial(
    pl.kernel,
    mesh=_mesh,
    out_type=jax.ShapeDtypeStruct((BATCH, 2 * LATENT_DIM), jnp.float32),
    scratch_types=(
        [pltpu.VMEM((BATCH,), jnp.int32)] * 5
        + [pltpu.VMEM((LATENT_DIM, _SB), jnp.float32)] * _RING
        + [pltpu.VMEM((LATENT_DIM, _TAIL), jnp.float32)]
        + [pltpu.VMEM((_L, 2 * LATENT_DIM), jnp.float32)]
        + [pltpu.VMEM((_L,), jnp.int32)]
        + [pltpu.SemaphoreType.DMA] * (_RING + 1)
    ),
    compiler_params=pltpu.CompilerParams(needs_layout_passes=False),
)
def _gather_sc(table_hbm, idx_hbm, out_hbm, idx_v, hit_i, hit_b, mi_v, mb_v,
               blk0, blk1, blk2, blk3, tail_v, sel_v, selb_v,
               gsem0, gsem1, gsem2, gsem3, wsem):
    rings = (blk0, blk1, blk2, blk3)
    gsems = (gsem0, gsem1, gsem2, gsem3)
    wid = lax.axis_index("s") * _NC + lax.axis_index("c")
    lane = lax.iota(jnp.int32, _L)

    pltpu.sync_copy(idx_hbm, idx_v)

    def scan_body(t, cnt):
        ivec = idx_v[pl.ds(t * _L, _L)]
        m = ((ivec >> 7) & (_NW - 1)) == wid
        plsc.store_compressed(hit_i.at[pl.ds(cnt, _L)], ivec, mask=m)
        plsc.store_compressed(hit_b.at[pl.ds(cnt, _L)], lane + t * _L, mask=m)
        return cnt + jnp.max(plsc.all_reduce_population_count(m))

    cnt = lax.fori_loop(0, BATCH // _L, scan_body, 0)
    nch = (cnt + _L - 1) >> 4

    def handle_block(sb, blk_ref):
        def match_body(t, mcnt):
            hv = hit_i[pl.ds(t * _L, _L)]
            hb = hit_b[pl.ds(t * _L, _L)]
            m2 = jnp.logical_and((hv >> 7) == sb, (lane + t * _L) < cnt)
            plsc.store_compressed(mi_v.at[pl.ds(mcnt, _L)], hv, mask=m2)
            plsc.store_compressed(mb_v.at[pl.ds(mcnt, _L)], hb, mask=m2)
            return mcnt + jnp.max(plsc.all_reduce_population_count(m2))

        mcnt = lax.fori_loop(0, nch, match_body, 0)

        def group_body(g, carry):
            mvec = mi_v[pl.ds(g * _L, _L)]
            bvec = mb_v[pl.ds(g * _L, _L)]
            valid = (lane + g * _L) < mcnt
            lvec = mvec & (_SB - 1)
            l0 = jnp.sum(jnp.where(lane == 0, lvec, 0))
            b0 = jnp.sum(jnp.where(lane == 0, bvec, 0))
            lvec = jnp.where(valid, lvec, l0)
            selb_v[...] = jnp.where(valid, bvec, b0)

            def dcol(d, c2):
                dv = jnp.full((_L,), d, jnp.int32)
                vals = plsc.load_gather(blk_ref, [dv, lvec])
                plsc.store_scatter(sel_v, [lane, dv], vals)
                return c2

            lax.fori_loop(0, LATENT_DIM, dcol, 0, unroll=8)
            pltpu.async_copy(sel_v, out_hbm.at[selb_v], wsem)
            pltpu.make_async_copy(sel_v, out_hbm.at[selb_v], wsem).wait()
            return carry

        lax.fori_loop(0, (mcnt + _L - 1) >> 4, group_body, 0)

    def fire(k, r):
        @pl.when(k < _FULL_K)
        def _():
            sb = wid + _NW * k
            pltpu.async_copy(table_hbm.at[:, pl.ds(sb * _SB, _SB)],
                             rings[r], gsems[r])

    for r in range(_RING - 1):
        fire(r, r)

    def stream_body(h, carry):
        for r in range(_RING):
            k = _RING * h + r
            pltpu.make_async_copy(table_hbm.at[:, pl.ds(0, _SB)], rings[r],
                                  gsems[r]).wait()
            fire(k + _RING - 1, (r + _RING - 1) % _RING)
            handle_block(wid + _NW * k, rings[r])
        return carry

    lax.fori_loop(0, _FULL_K // _RING, stream_body, 0)

    @pl.when(wid < _NSB - _FULL_K * _NW)
    def _():
        sb = wid + _FULL_K * _NW
        pltpu.sync_copy(table_hbm.at[:, pl.ds(sb * _SB, _SB)], blk0)
        handle_block(sb, blk0)

    @pl.when(wid == _NSB - _FULL_K * _NW)
    def _():
        pltpu.sync_copy(table_hbm.at[:, pl.ds(_NSB * _SB, _TAIL)], tail_v)
        handle_block(_NSB, tail_v)


def kernel(indices, latents):
    padded = _gather_sc(latents.T, indices.astype(jnp.int32))
    return padded[:, :LATENT_DIM]


# speedup vs baseline: 1.3904x; 1.1177x over previous
import functools

import jax
import jax.numpy as jnp
from jax import lax
from jax.experimental import pallas as pl
from jax.experimental.pallas import tpu as pltpu
from jax.experimental.pallas import tpu_sc as plsc

POOL_SIZE = 1000000
LATENT_DIM = 64
BATCH = 16384

_info = plsc.get_sparse_core_info()
_NC, _NS, _L = _info.num_cores, _info.num_subcores, _info.num_lanes
_NW = _NC * _NS
_SB = 128
_NSB = POOL_SIZE // _SB
_TAIL = POOL_SIZE - _NSB * _SB
_FULL_K = _NSB // _NW
_RING = 4

_mesh = plsc.VectorSubcoreMesh(core_axis_name="c", subcore_axis_name="s")


@functools.partial(
    pl.kernel,
    mesh=_mesh,
    out_type=jax.ShapeDtypeStruct((BATCH, 2 * LATENT_DIM), jnp.float32),
    scratch_types=(
        [pltpu.VMEM((BATCH,), jnp.int32)] * 5
        + [pltpu.VMEM((LATENT_DIM, _SB), jnp.float32)] * _RING
        + [pltpu.VMEM((LATENT_DIM, _TAIL), jnp.float32)]
        + [pltpu.VMEM((_L, 2 * LATENT_DIM), jnp.float32)] * 2
        + [pltpu.VMEM((_L,), jnp.int32)] * 2
        + [pltpu.SemaphoreType.DMA] * (_RING + 2)
    ),
    compiler_params=pltpu.CompilerParams(needs_layout_passes=False),
)
def _gather_sc(table_hbm, idx_hbm, out_hbm, idx_v, hit_i, hit_b, mi_v, mb_v,
               blk0, blk1, blk2, blk3, tail_v, sel0, sel1, selb0, selb1,
               gsem0, gsem1, gsem2, gsem3, wsem0, wsem1):
    rings = (blk0, blk1, blk2, blk3)
    gsems = (gsem0, gsem1, gsem2, gsem3)
    sels = (sel0, sel1)
    selbs = (selb0, selb1)
    wsems = (wsem0, wsem1)
    wid = lax.axis_index("s") * _NC + lax.axis_index("c")
    lane = lax.iota(jnp.int32, _L)

    pltpu.sync_copy(idx_hbm, idx_v)

    def scan_body(t, cnt):
        ivec = idx_v[pl.ds(t * _L, _L)]
        m = ((ivec >> 7) & (_NW - 1)) == wid
        plsc.store_compressed(hit_i.at[pl.ds(cnt, _L)], ivec, mask=m)
        plsc.store_compressed(hit_b.at[pl.ds(cnt, _L)], lane + t * _L, mask=m)
        return cnt + jnp.max(plsc.all_reduce_population_count(m))

    cnt = lax.fori_loop(0, BATCH // _L, scan_body, 0)
    nch = (cnt + _L - 1) >> 4

    def handle_block(sb, blk_ref, r, pending):
        sel_v, selb_v, wsem = sels[r % 2], selbs[r % 2], wsems[r % 2]
        def match_body(t, mcnt):
            hv = hit_i[pl.ds(t * _L, _L)]
            hb = hit_b[pl.ds(t * _L, _L)]
            m2 = jnp.logical_and((hv >> 7) == sb, (lane + t * _L) < cnt)
            plsc.store_compressed(mi_v.at[pl.ds(mcnt, _L)], hv, mask=m2)
            plsc.store_compressed(mb_v.at[pl.ds(mcnt, _L)], hb, mask=m2)
            return mcnt + jnp.max(plsc.all_reduce_population_count(m2))

        mcnt = lax.fori_loop(0, nch, match_body, 0)

        def group_body(g, carry):
            @pl.when(jnp.logical_or(g > 0, pending == 1))
            def _():
                pltpu.make_async_copy(sel_v, out_hbm.at[selb_v], wsem).wait()
            mvec = mi_v[pl.ds(g * _L, _L)]
            bvec = mb_v[pl.ds(g * _L, _L)]
            valid = (lane + g * _L) < mcnt
            lvec = mvec & (_SB - 1)
            l0 = jnp.sum(jnp.where(lane == 0, lvec, 0))
            b0 = jnp.sum(jnp.where(lane == 0, bvec, 0))
            lvec = jnp.where(valid, lvec, l0)
            selb_v[...] = jnp.where(valid, bvec, b0)

            def dcol(d, c2):
                dv = jnp.full((_L,), d, jnp.int32)
                vals = plsc.load_gather(blk_ref, [dv, lvec])
                plsc.store_scatter(sel_v, [lane, dv], vals)
                return c2

            lax.fori_loop(0, LATENT_DIM, dcol, 0, unroll=8)
            pltpu.async_copy(sel_v, out_hbm.at[selb_v], wsem)
            return carry

        ngroups = (mcnt + _L - 1) >> 4
        lax.fori_loop(0, ngroups, group_body, 0)
        return jnp.where(ngroups > 0, 1, pending)

    def fire(k, r):
        @pl.when(k < _FULL_K)
        def _():
            sb = wid + _NW * k
            pltpu.async_copy(table_hbm.at[:, pl.ds(sb * _SB, _SB)],
                             rings[r], gsems[r])

    for r in range(_RING - 1):
        fire(r, r)

    def stream_body(h, pend):
        pend = list(pend)
        for r in range(_RING):
            k = _RING * h + r
            pltpu.make_async_copy(table_hbm.at[:, pl.ds(0, _SB)], rings[r],
                                  gsems[r]).wait()
            fire(k + _RING - 1, (r + _RING - 1) % _RING)
            pend[r % 2] = handle_block(wid + _NW * k, rings[r], r, pend[r % 2])
        return tuple(pend)

    pend = lax.fori_loop(0, _FULL_K // _RING, stream_body,
                         (jnp.int32(0),) * 2)

    for r in range(2):
        @pl.when(pend[r] == 1)
        def _(r=r):
            pltpu.make_async_copy(sels[r], out_hbm.at[selbs[r]],
                                  wsems[r]).wait()

    @pl.when(wid < _NSB - _FULL_K * _NW)
    def _():
        sb = wid + _FULL_K * _NW
        pltpu.sync_copy(table_hbm.at[:, pl.ds(sb * _SB, _SB)], blk0)
        p = handle_block(sb, blk0, 0, jnp.int32(0))

        @pl.when(p == 1)
        def _():
            pltpu.make_async_copy(sel0, out_hbm.at[selb0], wsem0).wait()

    @pl.when(wid == _NSB - _FULL_K * _NW)
    def _():
        pltpu.sync_copy(table_hbm.at[:, pl.ds(_NSB * _SB, _TAIL)], tail_v)
        p = handle_block(_NSB, tail_v, 0, jnp.int32(0))

        @pl.when(p == 1)
        def _():
            pltpu.make_async_copy(sel0, out_hbm.at[selb0], wsem0).wait()


def kernel(indices, latents):
    padded = _gather_sc(latents.T, indices.astype(jnp.int32))
    return padded[:, :LATENT_DIM]


# speedup vs baseline: 1.8200x; 1.3089x over previous
import functools

import jax
import jax.numpy as jnp
from jax import lax
from jax.experimental import pallas as pl
from jax.experimental.pallas import tpu as pltpu
from jax.experimental.pallas import tpu_sc as plsc

POOL_SIZE = 1000000
LATENT_DIM = 64
BATCH = 16384

_info = plsc.get_sparse_core_info()
_NC, _NS, _L = _info.num_cores, _info.num_subcores, _info.num_lanes
_NW = _NC * _NS
_SB = 128
_NSB = POOL_SIZE // _SB
_TAIL = POOL_SIZE - _NSB * _SB
_FULL_K = _NSB // _NW
_RING = 4
_NBIN = 8

_mesh = plsc.VectorSubcoreMesh(core_axis_name="c", subcore_axis_name="s")


@functools.partial(
    pl.kernel,
    mesh=_mesh,
    out_type=jax.ShapeDtypeStruct((BATCH, 2 * LATENT_DIM), jnp.float32),
    scratch_types=(
        [pltpu.VMEM((BATCH,), jnp.int32)] * 5
        + [pltpu.VMEM((_L,), jnp.int32)]
        + [pltpu.VMEM((LATENT_DIM, _SB), jnp.float32)] * _RING
        + [pltpu.VMEM((LATENT_DIM, _TAIL), jnp.float32)]
        + [pltpu.VMEM((_L, 2 * LATENT_DIM), jnp.float32)] * 2
        + [pltpu.VMEM((_L,), jnp.int32)] * 2
        + [pltpu.SemaphoreType.DMA] * (_RING + 2)
    ),
    compiler_params=pltpu.CompilerParams(needs_layout_passes=False),
)
def _gather_sc(table_hbm, idx_hbm, out_hbm, idx_v, hit_i, hit_b,
               mi_v, mb_v, cnt_v, blk0, blk1, blk2, blk3, tail_v,
               sel0, sel1, selb0, selb1,
               gsem0, gsem1, gsem2, gsem3, wsem0, wsem1):
    bin_v = idx_v
    rings = (blk0, blk1, blk2, blk3)
    gsems = (gsem0, gsem1, gsem2, gsem3)
    sels = (sel0, sel1)
    selbs = (selb0, selb1)
    wsems = (wsem0, wsem1)
    wid = lax.axis_index("s") * _NC + lax.axis_index("c")
    lane = lax.iota(jnp.int32, _L)

    def popcnt(m):
        return plsc.all_reduce_population_count(m)[0]

    pltpu.sync_copy(idx_hbm, idx_v)

    def scan_body(t, cnt):
        ivec = idx_v[pl.ds(t * _L, _L)]
        m = ((ivec >> 7) & (_NW - 1)) == wid
        plsc.store_compressed(hit_i.at[pl.ds(cnt, _L)], ivec, mask=m)
        plsc.store_compressed(hit_b.at[pl.ds(cnt, _L)], lane + t * _L, mask=m)
        return cnt + popcnt(m)

    cnt = lax.fori_loop(0, BATCH // _L, scan_body, 0)
    nch = (cnt + _L - 1) >> 4

    offs = []
    cur = 0
    for jb in range(_NBIN):
        offs.append(cur)

        def bin_body(t, boff, jb=jb):
            hv = hit_i[pl.ds(t * _L, _L)]
            hb = hit_b[pl.ds(t * _L, _L)]
            rel = (hv >> 7) - wid
            m = jnp.logical_and((rel >> 10) == jb, (lane + t * _L) < cnt)
            packed = (((rel >> 5) & 31) << 21) | ((hv & 127) << 14) | hb
            plsc.store_compressed(bin_v.at[pl.ds(boff, _L)], packed, mask=m)
            return boff + popcnt(m)

        cur = lax.fori_loop(0, nch, bin_body, cur)
    offs.append(cur)

    def handle_block(kk, off, end, blk_ref, r, pending):
        sel_v, selb_v, wsem = sels[r % 2], selbs[r % 2], wsems[r % 2]

        def match_body(t, mcnt):
            pv = bin_v[pl.ds(t * _L, _L)]
            pos = lane + t * _L
            m2 = jnp.logical_and(
                (pv >> 21) == kk,
                jnp.logical_and(pos >= off, pos < end))
            plsc.store_compressed(mi_v.at[pl.ds(mcnt, _L)], (pv >> 14) & 127,
                                  mask=m2)
            plsc.store_compressed(mb_v.at[pl.ds(mcnt, _L)], pv & 0x3FFF,
                                  mask=m2)
            return mcnt + popcnt(m2)

        mcnt = lax.fori_loop(off >> 4, (end + _L - 1) >> 4, match_body, 0)

        def group_body(g, carry):
            @pl.when(jnp.logical_or(g > 0, pending == 1))
            def _():
                pltpu.make_async_copy(sel_v, out_hbm.at[selb_v], wsem).wait()
            lvec = mi_v[pl.ds(g * _L, _L)]
            bvec = mb_v[pl.ds(g * _L, _L)]
            valid = (lane + g * _L) < mcnt
            l0 = jnp.sum(jnp.where(lane == 0, lvec, 0))
            b0 = jnp.sum(jnp.where(lane == 0, bvec, 0))
            lvec = jnp.where(valid, lvec, l0)
            selb_v[...] = jnp.where(valid, bvec, b0)

            def dcol(d, c2):
                dv = jnp.full((_L,), d, jnp.int32)
                vals = plsc.load_gather(blk_ref, [dv, lvec])
                plsc.store_scatter(sel_v, [lane, dv], vals)
                return c2

            lax.fori_loop(0, LATENT_DIM, dcol, 0, unroll=8)
            pltpu.async_copy(sel_v, out_hbm.at[selb_v], wsem)
            return carry

        ngroups = (mcnt + _L - 1) >> 4
        lax.fori_loop(0, ngroups, group_body, 0)
        return jnp.where(ngroups > 0, 1, pending)

    def fire(k, r):
        @pl.when(k < _FULL_K)
        def _():
            sb = wid + _NW * k
            pltpu.async_copy(table_hbm.at[:, pl.ds(sb * _SB, _SB)],
                             rings[r], gsems[r])

    for r in range(_RING - 1):
        fire(r, r)

    pend = (jnp.int32(0),) * 2
    for jb in range(_NBIN):
        h0, h1 = 8 * jb, min(8 * (jb + 1), _FULL_K // _RING)

        def stream_body(h, pend, jb=jb):
            pend = list(pend)
            for r in range(_RING):
                k = _RING * h + r
                pltpu.make_async_copy(table_hbm.at[:, pl.ds(0, _SB)],
                                      rings[r], gsems[r]).wait()
                fire(k + _RING - 1, (r + _RING - 1) % _RING)
                pend[r % 2] = handle_block(k & 31, offs[jb], offs[jb + 1],
                                           rings[r], r, pend[r % 2])
            return tuple(pend)

        pend = lax.fori_loop(h0, h1, stream_body, pend)

    for r in range(2):
        @pl.when(pend[r] == 1)
        def _(r=r):
            pltpu.make_async_copy(sels[r], out_hbm.at[selbs[r]],
                                  wsems[r]).wait()

    @pl.when(wid < _NSB - _FULL_K * _NW)
    def _():
        sb = wid + _FULL_K * _NW
        pltpu.sync_copy(table_hbm.at[:, pl.ds(sb * _SB, _SB)], blk0)
        p = handle_block(_FULL_K & 31, offs[_NBIN - 1], offs[_NBIN],
                         blk0, 0, jnp.int32(0))

        @pl.when(p == 1)
        def _():
            pltpu.make_async_copy(sel0, out_hbm.at[selb0], wsem0).wait()

    @pl.when(wid == _NSB - _FULL_K * _NW)
    def _():
        pltpu.sync_copy(table_hbm.at[:, pl.ds(_NSB * _SB, _TAIL)], tail_v)
        p = handle_block(_FULL_K & 31, offs[_NBIN - 1], offs[_NBIN],
                         tail_v, 0, jnp.int32(0))

        @pl.when(p == 1)
        def _():
            pltpu.make_async_copy(sel0, out_hbm.at[selb0], wsem0).wait()


def kernel(indices, latents):
    padded = _gather_sc(latents.T, indices.astype(jnp.int32))
    return padded[:, :LATENT_DIM]
